# baseline (speedup 1.0000x reference)
import jax
import jax.numpy as jnp
from jax.experimental import pallas as pl
from jax.experimental.pallas import tpu as pltpu

_R = 4
_NB = 8
_ROUNDS = 4


def _dma_kernel(tid_ref, ep_ref, out_ref, sel_buf, gsem, *bufs_and_sems):
    bufs = bufs_and_sems[:_NB]
    sems = bufs_and_sems[_NB:]
    tid = tid_ref[0]
    gcp = pltpu.make_async_copy(ep_ref.at[:, tid], sel_buf, gsem)
    gcp.start()
    gcp.wait()
    src = sel_buf[...][:, None]
    for k in range(_NB):
        bufs[k][...] = jnp.broadcast_to(src, bufs[k].shape)
    for r in range(_ROUNDS):
        for k in range(_NB):
            off = (r * _NB + k) * _R
            pltpu.make_async_copy(
                bufs[k], out_ref.at[:, off:off + _R], sems[k]
            ).start()
        for k in range(_NB):
            off = (r * _NB + k) * _R
            pltpu.make_async_copy(
                bufs[k], out_ref.at[:, off:off + _R], sems[k]
            ).wait()


def kernel(x_query, vis_mark, e_p, task_id):
    del vis_mark
    B = x_query.shape[0]
    nL, _, P, D = e_p.shape
    assert B == _R * _NB * _ROUNDS
    tid = jnp.asarray(task_id, jnp.int32).reshape((1,))
    scratch = [pltpu.VMEM((nL, P, D), jnp.float32), pltpu.SemaphoreType.DMA]
    scratch += [pltpu.VMEM((nL, _R, P, D), jnp.float32) for _ in range(_NB)]
    scratch += [pltpu.SemaphoreType.DMA for _ in range(_NB)]
    return pl.pallas_call(
        _dma_kernel,
        grid_spec=pltpu.PrefetchScalarGridSpec(
            num_scalar_prefetch=1,
            grid=(1,),
            in_specs=[pl.BlockSpec(memory_space=pl.ANY)],
            out_specs=pl.BlockSpec(memory_space=pl.ANY),
            scratch_shapes=scratch,
        ),
        out_shape=jax.ShapeDtypeStruct((nL, B, P, D), e_p.dtype),
    )(tid, e_p)
